# baseline (speedup 1.0000x reference)
import functools

import jax
import jax.numpy as jnp
from jax import lax
from jax.experimental import pallas as pl
from jax.experimental.pallas import tpu as pltpu
from jax.experimental.pallas import tpu_sc as plsc

F = 26
V = 100000
D = 16
B = 4096

NC = 2
NS = 16
NW = NC * NS
R = F * D

NCOL = 1024
NSTEP = B // NCOL


def _gather_body(tbl_hbm, idx_hbm, out_hbm, idx_v, rows_v, sem):
    wid = lax.axis_index("s") * NC + lax.axis_index("c")

    @pl.when(wid < F)
    def _():
        f = wid
        pltpu.sync_copy(idx_hbm.at[f], idx_v)
        copies = []
        for s in range(D):
            copies.append(
                pltpu.async_copy(tbl_hbm.at[f * D + s].at[idx_v],
                                 rows_v.at[s], sem))
        for c in copies:
            c.wait()
        pltpu.sync_copy(rows_v, out_hbm.at[f])


@functools.lru_cache(maxsize=1)
def _gather():
    return functools.partial(
        pl.kernel,
        mesh=plsc.VectorSubcoreMesh(core_axis_name="c", subcore_axis_name="s"),
        compiler_params=pltpu.CompilerParams(use_tc_tiling_on_sc=False),
        out_type=jax.ShapeDtypeStruct((F, D, B), jnp.float32),
        scratch_types=[
            pltpu.VMEM((B,), jnp.int32),
            pltpu.VMEM((D, B), jnp.float32),
            pltpu.SemaphoreType.DMA,
        ],
    )(_gather_body)


def _fwfm_body(e_ref, a_ref, w_ref, o_ref):
    a = a_ref[...]
    acc = jnp.zeros((1, B), jnp.float32)
    for d in range(D):
        e = e_ref[:, d, :]
        p = jnp.dot(a, e, preferred_element_type=jnp.float32)
        y = e * (p + w_ref[:, d][:, None])
        acc = acc + jnp.sum(y, axis=0, keepdims=True)
    o_ref[...] = jax.nn.sigmoid(acc)


def kernel(x, tables, field_cov, fwfm_linear_w):
    m = jnp.transpose(tables, (0, 2, 1)).reshape(R, V)
    xt = x.T.astype(jnp.int32)

    et = _gather()(m, xt)

    sym = (field_cov + field_cov.T) * 0.5
    a_half = 0.5 * (sym - jnp.diag(jnp.diag(sym)))

    out2 = pl.pallas_call(
        _fwfm_body,
        in_specs=[
            pl.BlockSpec((F, D, B), lambda: (0, 0, 0)),
            pl.BlockSpec((F, F), lambda: (0, 0)),
            pl.BlockSpec((F, D), lambda: (0, 0)),
        ],
        out_specs=pl.BlockSpec((1, B), lambda: (0, 0)),
        out_shape=jax.ShapeDtypeStruct((1, B), jnp.float32),
    )(et, a_half, fwfm_linear_w)
    return out2.reshape(B)
